# baseline (speedup 1.0000x reference)
import dataclasses
import functools

import jax
import jax.numpy as jnp
from jax import lax
from jax.experimental import pallas as pl
from jax.experimental.pallas import tpu as pltpu
from jax.experimental.pallas import tpu_sc as plsc

B = 16
N = 20000
L = 16
W = 48

_LIM0 = 63.0
_LIM12 = 255.0
_SCALE = 0.1


def _row_work(b, cls_hbm, reg_hbm, anc_hbm, scores_hbm, boxes_hbm,
              row_v, rg_v, an_v, tmp_v, sco_v, box_v, sem_r, sem_a):
    pltpu.sync_copy(cls_hbm.at[pl.ds(b * N, N)], row_v)

    lane = lax.iota(jnp.int32, L)

    def body(i, carry):
        bv, bi = carry
        v = row_v[pl.ds(i * L, L)]
        m = v > bv
        return jnp.where(m, v, bv), jnp.where(m, lane + i * L, bi)

    init = (row_v[pl.ds(0, L)], lane)
    bv, bi = lax.fori_loop(1, N // L, body, init, unroll=4)

    smax = jnp.max(bv)
    cand = jnp.where(bv == smax, bi, jnp.int32(2**31 - 1))
    idx = jnp.min(cand)

    start = idx * 6
    aligned = jnp.minimum((start // L) * L, N * 6 - W)
    off = start - aligned
    cp_r = pltpu.make_async_copy(
        reg_hbm.at[pl.ds(b * (N * 6) + aligned, W)], rg_v.at[pl.ds(0, W)], sem_r)
    cp_a = pltpu.make_async_copy(
        anc_hbm.at[pl.ds(b * (N * 6) + aligned, W)], an_v.at[pl.ds(0, W)], sem_a)
    cp_r.start()
    cp_a.start()
    cp_r.wait()
    cp_a.wait()

    a0 = an_v[pl.ds(off, L)]
    a1 = an_v[pl.ds(off + 3, L)]
    r0 = rg_v[pl.ds(off, L)] * _SCALE
    r1 = rg_v[pl.ds(off + 3, L)] * _SCALE

    dhw = a1 - a0
    ctr = a0 + 0.5 * dhw
    pdhw = jnp.exp(r1) * dhw
    pctr = r0 * dhw + ctr
    mins = jnp.maximum(pctr - 0.5 * pdhw, 0.0)
    lim = jnp.where(lane == 0, _LIM0, _LIM12)
    maxs = jnp.minimum(pctr + 0.5 * pdhw, lim)

    tmp_v[pl.ds(3, L)] = maxs
    box_v[...] = jnp.where(lane < 3, mins, tmp_v[pl.ds(0, L)])
    sco_v[...] = jnp.zeros((L,), jnp.float32) + smax
    pltpu.sync_copy(sco_v, scores_hbm.at[pl.ds(b * L, L)])
    pltpu.sync_copy(box_v, boxes_hbm.at[pl.ds(b * L, L)])


def _sc_body(cls_hbm, reg_hbm, anc_hbm, scores_hbm, boxes_hbm,
             row_v, rg_v, an_v, tmp_v, sco_v, box_v, sem_r, sem_a):
    c = lax.axis_index("c")
    s = lax.axis_index("s")

    @pl.when(c == 0)
    def _():
        _row_work(s, cls_hbm, reg_hbm, anc_hbm, scores_hbm, boxes_hbm,
                  row_v, rg_v, an_v, tmp_v, sco_v, box_v, sem_r, sem_a)


@jax.jit
def _sc_decode(cls_flat, reg_flat, anc_flat):
    mesh = plsc.VectorSubcoreMesh(
        core_axis_name="c", subcore_axis_name="s", num_cores=2, num_subcores=16
    )
    cp = pltpu.CompilerParams()
    if "needs_layout_passes" in pltpu.CompilerParams.__dataclass_fields__:
        cp = dataclasses.replace(cp, needs_layout_passes=False)
    run = functools.partial(
        pl.kernel,
        compiler_params=cp,
        out_type=(
            jax.ShapeDtypeStruct((B * L,), jnp.float32),
            jax.ShapeDtypeStruct((B * L,), jnp.float32),
        ),
        mesh=mesh,
        scratch_types=[
            pltpu.VMEM((N,), jnp.float32),
            pltpu.VMEM((W + L,), jnp.float32),
            pltpu.VMEM((W + L,), jnp.float32),
            pltpu.VMEM((L + 3,), jnp.float32),
            pltpu.VMEM((L,), jnp.float32),
            pltpu.VMEM((L,), jnp.float32),
            pltpu.SemaphoreType.DMA,
            pltpu.SemaphoreType.DMA,
        ],
    )(_sc_body)
    return run(cls_flat, reg_flat, anc_flat)


def kernel(cls_heads, reg_heads, batch_anchors):
    cls_flat = cls_heads.reshape(B * N)
    reg_flat = reg_heads.reshape(B * N * 6)
    anc_flat = batch_anchors.reshape(B * N * 6)
    scores_full, boxes_full = _sc_decode(cls_flat, reg_flat, anc_flat)
    scores_full = scores_full.reshape(B, L)
    boxes_full = boxes_full.reshape(B, L)
    return scores_full[:, 0], boxes_full[:, :6]


# speedup vs baseline: 1.8405x; 1.8405x over previous
import dataclasses
import functools

import jax
import jax.numpy as jnp
from jax import lax
from jax.experimental import pallas as pl
from jax.experimental.pallas import tpu as pltpu
from jax.experimental.pallas import tpu_sc as plsc

B = 16
N = 20000
L = 16

_LIM = (63.0, 255.0, 255.0)
_SCALE = 0.1


def _row_work(b, cls_hbm, scores_hbm, idx_hbm, row_v, sco_v, idx_v):
    pltpu.sync_copy(cls_hbm.at[pl.ds(b * N, N)], row_v)

    lane = lax.iota(jnp.int32, L)

    def body(i, carry):
        bv, bi = carry
        v = row_v[pl.ds(i * L, L)]
        m = v > bv
        return jnp.where(m, v, bv), jnp.where(m, lane + i * L, bi)

    init = (row_v[pl.ds(0, L)], lane)
    bv, bi = lax.fori_loop(1, N // L, body, init, unroll=4)

    smax = jnp.max(bv)
    cand = jnp.where(bv == smax, bi, jnp.int32(2**31 - 1))
    idx = jnp.min(cand)

    sco_v[...] = jnp.zeros((L,), jnp.float32) + smax
    idx_v[...] = jnp.zeros((L,), jnp.int32) + idx
    pltpu.sync_copy(sco_v, scores_hbm.at[pl.ds(b * L, L)])
    pltpu.sync_copy(idx_v, idx_hbm.at[pl.ds(b * L, L)])


def _sc_body(cls_hbm, scores_hbm, idx_hbm, row_v, sco_v, idx_v):
    c = lax.axis_index("c")
    s = lax.axis_index("s")

    @pl.when(c == 0)
    def _():
        _row_work(s, cls_hbm, scores_hbm, idx_hbm, row_v, sco_v, idx_v)


def _tc_body(idx_smem, reg_any, anc_any, box_out, rg_s, an_s, sem_r, sem_a):
    cps = []
    for b in range(B):
        ib = idx_smem[b * L]
        cps.append(pltpu.make_async_copy(
            reg_any.at[b, ib], rg_s.at[b], sem_r))
        cps.append(pltpu.make_async_copy(
            anc_any.at[b, ib], an_s.at[b], sem_a))
    for cp in cps:
        cp.start()
    for cp in cps:
        cp.wait()

    rg = rg_s[...] * _SCALE
    an = an_s[...]
    a_lo = an[:, 0:3]
    a_hi = an[:, 3:6]
    dhw = a_hi - a_lo
    ctr = a_lo + 0.5 * dhw
    pdhw = jnp.exp(rg[:, 3:6]) * dhw
    pctr = rg[:, 0:3] * dhw + ctr
    mins = jnp.maximum(pctr - 0.5 * pdhw, 0.0)
    col = lax.broadcasted_iota(jnp.int32, (B, 3), 1)
    lim = jnp.where(col == 0, _LIM[0], _LIM[1])
    maxs = jnp.minimum(pctr + 0.5 * pdhw, lim)
    box_out[:, 0:3] = mins
    box_out[:, 3:6] = maxs


@jax.jit
def _run(cls_flat, reg_heads, batch_anchors):
    mesh = plsc.VectorSubcoreMesh(
        core_axis_name="c", subcore_axis_name="s", num_cores=2, num_subcores=16
    )
    cp = pltpu.CompilerParams()
    if "needs_layout_passes" in pltpu.CompilerParams.__dataclass_fields__:
        cp = dataclasses.replace(cp, needs_layout_passes=False)
    sc_argmax = functools.partial(
        pl.kernel,
        compiler_params=cp,
        out_type=(
            jax.ShapeDtypeStruct((B * L,), jnp.float32),
            jax.ShapeDtypeStruct((B * L,), jnp.int32),
        ),
        mesh=mesh,
        scratch_types=[
            pltpu.VMEM((N,), jnp.float32),
            pltpu.VMEM((L,), jnp.float32),
            pltpu.VMEM((L,), jnp.int32),
        ],
    )(_sc_body)
    scores_full, idx_full = sc_argmax(cls_flat)

    boxes = pl.pallas_call(
        _tc_body,
        out_shape=jax.ShapeDtypeStruct((B, 6), jnp.float32),
        in_specs=[
            pl.BlockSpec(memory_space=pltpu.SMEM),
            pl.BlockSpec(memory_space=pl.ANY),
            pl.BlockSpec(memory_space=pl.ANY),
        ],
        out_specs=pl.BlockSpec(memory_space=pltpu.VMEM),
        scratch_shapes=[
            pltpu.VMEM((B, 6), jnp.float32),
            pltpu.VMEM((B, 6), jnp.float32),
            pltpu.SemaphoreType.DMA,
            pltpu.SemaphoreType.DMA,
        ],
    )(idx_full, reg_heads, batch_anchors)

    scores = scores_full.reshape(B, L)[:, 0]
    return scores, boxes


def kernel(cls_heads, reg_heads, batch_anchors):
    return _run(cls_heads.reshape(B * N), reg_heads, batch_anchors)


# speedup vs baseline: 11.1870x; 6.0783x over previous
import dataclasses
import functools

import jax
import jax.numpy as jnp
from jax import lax
from jax.experimental import pallas as pl
from jax.experimental.pallas import tpu as pltpu
from jax.experimental.pallas import tpu_sc as plsc

B = 16
N = 20000
L = 16
WIN = 128

_LIM = (63.0, 255.0, 255.0)
_SCALE = 0.1


def _row_work(b, cls_hbm, scores_hbm, idx_hbm, row_v, sco_v, idx_v):
    pltpu.sync_copy(cls_hbm.at[pl.ds(b * N, N)], row_v)

    lane = lax.iota(jnp.int32, L)

    def body(i, carry):
        bv, bi = carry
        v = row_v[pl.ds(i * L, L)]
        m = v > bv
        return jnp.where(m, v, bv), jnp.where(m, lane + i * L, bi)

    init = (row_v[pl.ds(0, L)], lane)
    bv, bi = lax.fori_loop(1, N // L, body, init, unroll=4)

    smax = jnp.max(bv)
    cand = jnp.where(bv == smax, bi, jnp.int32(2**31 - 1))
    idx = jnp.min(cand)

    sco_v[...] = jnp.zeros((L,), jnp.float32) + smax
    idx_v[...] = jnp.zeros((L,), jnp.int32) + idx
    pltpu.sync_copy(sco_v, scores_hbm.at[pl.ds(b * L, L)])
    pltpu.sync_copy(idx_v, idx_hbm.at[pl.ds(b * L, L)])


def _sc_body(cls_hbm, scores_hbm, idx_hbm, row_v, sco_v, idx_v):
    c = lax.axis_index("c")
    s = lax.axis_index("s")

    @pl.when(c == 0)
    def _():
        _row_work(s, cls_hbm, scores_hbm, idx_hbm, row_v, sco_v, idx_v)


def _tc_body(idx_smem, idx_vmem, reg_any, anc_any, box_out,
             rg_s, an_s, sem_r, sem_a):
    cps = []
    for b in range(B):
        ib = idx_smem[b * L]
        w = (ib // WIN) * WIN
        cps.append(pltpu.make_async_copy(
            reg_any.at[:, pl.ds((b // 8) * 8, 8), pl.ds(w, WIN)],
            rg_s.at[b], sem_r))
        cps.append(pltpu.make_async_copy(
            anc_any.at[:, pl.ds((b // 8) * 8, 8), pl.ds(w, WIN)],
            an_s.at[b], sem_a))
    for cp in cps:
        cp.start()
    for cp in cps:
        cp.wait()

    idxv = idx_vmem[:, :1]
    off = (idxv % WIN)[:, :, None, None]
    bmod = (lax.broadcasted_iota(jnp.int32, (B, 1, 1, 1), 0)) % 8
    rowi = lax.broadcasted_iota(jnp.int32, (B, 1, 8, WIN), 2)
    lanei = lax.broadcasted_iota(jnp.int32, (B, 1, 8, WIN), 3)
    mask = (rowi == bmod) & (lanei == off)
    rg = jnp.where(mask, rg_s[...], 0.0).sum(axis=3).sum(axis=2) * _SCALE
    an = jnp.where(mask, an_s[...], 0.0).sum(axis=3).sum(axis=2)

    a_lo = an[:, 0:3]
    a_hi = an[:, 3:6]
    dhw = a_hi - a_lo
    ctr = a_lo + 0.5 * dhw
    pdhw = jnp.exp(rg[:, 3:6]) * dhw
    pctr = rg[:, 0:3] * dhw + ctr
    mins = jnp.maximum(pctr - 0.5 * pdhw, 0.0)
    col = lax.broadcasted_iota(jnp.int32, (B, 3), 1)
    lim = jnp.where(col == 0, _LIM[0], _LIM[1])
    maxs = jnp.minimum(pctr + 0.5 * pdhw, lim)
    box_out[:, 0:3] = mins
    box_out[:, 3:6] = maxs


@jax.jit
def _run(cls_flat, reg_t, anc_t):
    mesh = plsc.VectorSubcoreMesh(
        core_axis_name="c", subcore_axis_name="s", num_cores=2, num_subcores=16
    )
    cp = pltpu.CompilerParams()
    if "needs_layout_passes" in pltpu.CompilerParams.__dataclass_fields__:
        cp = dataclasses.replace(cp, needs_layout_passes=False)
    sc_argmax = functools.partial(
        pl.kernel,
        compiler_params=cp,
        out_type=(
            jax.ShapeDtypeStruct((B * L,), jnp.float32),
            jax.ShapeDtypeStruct((B * L,), jnp.int32),
        ),
        mesh=mesh,
        scratch_types=[
            pltpu.VMEM((N,), jnp.float32),
            pltpu.VMEM((L,), jnp.float32),
            pltpu.VMEM((L,), jnp.int32),
        ],
    )(_sc_body)
    scores_full, idx_full = sc_argmax(cls_flat)

    boxes = pl.pallas_call(
        _tc_body,
        out_shape=jax.ShapeDtypeStruct((B, 6), jnp.float32),
        in_specs=[
            pl.BlockSpec(memory_space=pltpu.SMEM),
            pl.BlockSpec(memory_space=pltpu.VMEM),
            pl.BlockSpec(memory_space=pl.ANY),
            pl.BlockSpec(memory_space=pl.ANY),
        ],
        out_specs=pl.BlockSpec(memory_space=pltpu.VMEM),
        scratch_shapes=[
            pltpu.VMEM((B, 6, 8, WIN), jnp.float32),
            pltpu.VMEM((B, 6, 8, WIN), jnp.float32),
            pltpu.SemaphoreType.DMA,
            pltpu.SemaphoreType.DMA,
        ],
    )(idx_full, idx_full.reshape(B, L), reg_t, anc_t)

    scores = scores_full.reshape(B, L)[:, 0]
    return scores, boxes


def kernel(cls_heads, reg_heads, batch_anchors):
    reg_t = jnp.transpose(reg_heads, (2, 0, 1))
    anc_t = jnp.transpose(batch_anchors, (2, 0, 1))
    return _run(cls_heads.reshape(B * N), reg_t, anc_t)


# speedup vs baseline: 11.3532x; 1.0149x over previous
import dataclasses
import functools

import jax
import jax.numpy as jnp
from jax import lax
from jax.experimental import pallas as pl
from jax.experimental.pallas import tpu as pltpu
from jax.experimental.pallas import tpu_sc as plsc

B = 16
N = 20000
L = 16
WIN = 128

_LIM = (63.0, 255.0, 255.0)
_SCALE = 0.1


def _row_work(b, cls_hbm, scores_hbm, idx_hbm, row_v, sco_v, idx_v):
    pltpu.sync_copy(cls_hbm.at[pl.ds(b * N, N)], row_v)

    lane = lax.iota(jnp.int32, L)

    CH = 5

    def body(i, carry):
        out = []
        for k in range(CH):
            bv, bi = carry[2 * k], carry[2 * k + 1]
            v = row_v[pl.ds((i * CH + k) * L, L)]
            m = v > bv
            out.append(jnp.where(m, v, bv))
            out.append(jnp.where(m, lane + (i * CH + k) * L, bi))
        return tuple(out)

    init = []
    for k in range(CH):
        init.append(row_v[pl.ds(k * L, L)])
        init.append(lane + k * L)
    acc = lax.fori_loop(1, (N // L) // CH, body, tuple(init), unroll=2)

    bv, bi = acc[0], acc[1]
    for k in range(1, CH):
        bvk, bik = acc[2 * k], acc[2 * k + 1]
        m = (bvk > bv) | ((bvk == bv) & (bik < bi))
        bv = jnp.where(m, bvk, bv)
        bi = jnp.where(m, bik, bi)

    smax = jnp.max(bv)
    cand = jnp.where(bv == smax, bi, jnp.int32(2**31 - 1))
    idx = jnp.min(cand)

    sco_v[...] = jnp.zeros((L,), jnp.float32) + smax
    idx_v[...] = jnp.zeros((L,), jnp.int32) + idx
    pltpu.sync_copy(sco_v, scores_hbm.at[pl.ds(b * L, L)])
    pltpu.sync_copy(idx_v, idx_hbm.at[pl.ds(b * L, L)])


def _sc_body(cls_hbm, scores_hbm, idx_hbm, row_v, sco_v, idx_v):
    c = lax.axis_index("c")
    s = lax.axis_index("s")

    @pl.when(c == 0)
    def _():
        _row_work(s, cls_hbm, scores_hbm, idx_hbm, row_v, sco_v, idx_v)


def _tc_body(idx_smem, idx_vmem, reg_any, anc_any, box_out,
             rg_s, an_s, sem_r, sem_a):
    cps = []
    for b in range(B):
        ib = idx_smem[b * L]
        w = (ib // WIN) * WIN
        cps.append(pltpu.make_async_copy(
            reg_any.at[:, pl.ds((b // 8) * 8, 8), pl.ds(w, WIN)],
            rg_s.at[b], sem_r))
        cps.append(pltpu.make_async_copy(
            anc_any.at[:, pl.ds((b // 8) * 8, 8), pl.ds(w, WIN)],
            an_s.at[b], sem_a))
    for cp in cps:
        cp.start()
    for cp in cps:
        cp.wait()

    idxv = idx_vmem[:, :1]
    off = (idxv % WIN)[:, :, None, None]
    bmod = (lax.broadcasted_iota(jnp.int32, (B, 1, 1, 1), 0)) % 8
    rowi = lax.broadcasted_iota(jnp.int32, (B, 1, 8, WIN), 2)
    lanei = lax.broadcasted_iota(jnp.int32, (B, 1, 8, WIN), 3)
    mask = (rowi == bmod) & (lanei == off)
    rg = jnp.where(mask, rg_s[...], 0.0).sum(axis=3).sum(axis=2) * _SCALE
    an = jnp.where(mask, an_s[...], 0.0).sum(axis=3).sum(axis=2)

    a_lo = an[:, 0:3]
    a_hi = an[:, 3:6]
    dhw = a_hi - a_lo
    ctr = a_lo + 0.5 * dhw
    pdhw = jnp.exp(rg[:, 3:6]) * dhw
    pctr = rg[:, 0:3] * dhw + ctr
    mins = jnp.maximum(pctr - 0.5 * pdhw, 0.0)
    col = lax.broadcasted_iota(jnp.int32, (B, 3), 1)
    lim = jnp.where(col == 0, _LIM[0], _LIM[1])
    maxs = jnp.minimum(pctr + 0.5 * pdhw, lim)
    box_out[:, 0:3] = mins
    box_out[:, 3:6] = maxs


@jax.jit
def _run(cls_flat, reg_t, anc_t):
    mesh = plsc.VectorSubcoreMesh(
        core_axis_name="c", subcore_axis_name="s", num_cores=2, num_subcores=16
    )
    cp = pltpu.CompilerParams()
    if "needs_layout_passes" in pltpu.CompilerParams.__dataclass_fields__:
        cp = dataclasses.replace(cp, needs_layout_passes=False)
    sc_argmax = functools.partial(
        pl.kernel,
        compiler_params=cp,
        out_type=(
            jax.ShapeDtypeStruct((B * L,), jnp.float32),
            jax.ShapeDtypeStruct((B * L,), jnp.int32),
        ),
        mesh=mesh,
        scratch_types=[
            pltpu.VMEM((N,), jnp.float32),
            pltpu.VMEM((L,), jnp.float32),
            pltpu.VMEM((L,), jnp.int32),
        ],
    )(_sc_body)
    scores_full, idx_full = sc_argmax(cls_flat)

    boxes = pl.pallas_call(
        _tc_body,
        out_shape=jax.ShapeDtypeStruct((B, 6), jnp.float32),
        in_specs=[
            pl.BlockSpec(memory_space=pltpu.SMEM),
            pl.BlockSpec(memory_space=pltpu.VMEM),
            pl.BlockSpec(memory_space=pl.ANY),
            pl.BlockSpec(memory_space=pl.ANY),
        ],
        out_specs=pl.BlockSpec(memory_space=pltpu.VMEM),
        scratch_shapes=[
            pltpu.VMEM((B, 6, 8, WIN), jnp.float32),
            pltpu.VMEM((B, 6, 8, WIN), jnp.float32),
            pltpu.SemaphoreType.DMA,
            pltpu.SemaphoreType.DMA,
        ],
    )(idx_full, idx_full.reshape(B, L), reg_t, anc_t)

    scores = scores_full.reshape(B, L)[:, 0]
    return scores, boxes


def kernel(cls_heads, reg_heads, batch_anchors):
    reg_t = jnp.transpose(reg_heads, (2, 0, 1))
    anc_t = jnp.transpose(batch_anchors, (2, 0, 1))
    return _run(cls_heads.reshape(B * N), reg_t, anc_t)


# speedup vs baseline: 13.1111x; 1.1548x over previous
import dataclasses
import functools

import jax
import jax.numpy as jnp
from jax import lax
from jax.experimental import pallas as pl
from jax.experimental.pallas import tpu as pltpu
from jax.experimental.pallas import tpu_sc as plsc

B = 16
N = 20000
L = 16
WIN = 128
CH = 5

_LIM = (63.0, 255.0, 255.0)
_SCALE = 0.1


def _row_work(b, cls_hbm, scores_hbm, idx_hbm, row_v, sco_v, idx_v):
    pltpu.sync_copy(cls_hbm.at[pl.ds(b * N, N)], row_v)

    lane = lax.iota(jnp.int32, L)

    def body(i, carry):
        out = []
        for k in range(CH):
            bv, bi = carry[2 * k], carry[2 * k + 1]
            v = row_v[pl.ds((i * CH + k) * L, L)]
            m = v > bv
            out.append(jnp.where(m, v, bv))
            out.append(jnp.where(m, lane + (i * CH + k) * L, bi))
        return tuple(out)

    init = []
    for k in range(CH):
        init.append(row_v[pl.ds(k * L, L)])
        init.append(lane + k * L)
    acc = lax.fori_loop(1, (N // L) // CH, body, tuple(init), unroll=1)

    bv, bi = acc[0], acc[1]
    for k in range(1, CH):
        bvk, bik = acc[2 * k], acc[2 * k + 1]
        m = (bvk > bv) | ((bvk == bv) & (bik < bi))
        bv = jnp.where(m, bvk, bv)
        bi = jnp.where(m, bik, bi)

    smax = jnp.max(bv)
    cand = jnp.where(bv == smax, bi, jnp.int32(2**31 - 1))
    idx = jnp.min(cand)

    sco_v[...] = jnp.zeros((L,), jnp.float32) + smax
    idx_v[...] = jnp.zeros((L,), jnp.int32) + idx
    pltpu.sync_copy(sco_v, scores_hbm.at[pl.ds(b * L, L)])
    pltpu.sync_copy(idx_v, idx_hbm.at[pl.ds(b * L, L)])


def _sc_body(cls_hbm, scores_hbm, idx_hbm, row_v, sco_v, idx_v):
    c = lax.axis_index("c")
    s = lax.axis_index("s")

    @pl.when(s < 8)
    def _():
        _row_work(2 * s + c, cls_hbm, scores_hbm, idx_hbm,
                  row_v, sco_v, idx_v)


def _tc_body(idx_smem, sco_smem, reg_any, anc_any, sco_out, box_out,
             rg_s, an_s, sem_r, sem_a):
    cps = []
    for b in range(B):
        ib = idx_smem[b * L]
        w = (ib // WIN) * WIN
        cps.append(pltpu.make_async_copy(
            reg_any.at[:, pl.ds((b // 8) * 8, 8), pl.ds(w, WIN)],
            rg_s.at[:, b], sem_r))
        cps.append(pltpu.make_async_copy(
            anc_any.at[:, pl.ds((b // 8) * 8, 8), pl.ds(w, WIN)],
            an_s.at[:, b], sem_a))
    for cp in cps:
        cp.start()

    rows4 = lax.broadcasted_iota(jnp.int32, (1, B, 1, 1), 1)
    off = jnp.zeros((1, B, 1, 1), jnp.int32)
    lane1 = lax.broadcasted_iota(jnp.int32, (B,), 0)
    sco = jnp.zeros((B,), jnp.float32)
    for b in range(B):
        off = jnp.where(rows4 == b, idx_smem[b * L] % WIN, off)
        sco = jnp.where(lane1 == b, sco_smem[b * L], sco)
    sco_out[...] = sco

    for cp in cps:
        cp.wait()

    rowi = lax.broadcasted_iota(jnp.int32, (1, B, 8, WIN), 2)
    lanei = lax.broadcasted_iota(jnp.int32, (1, B, 8, WIN), 3)
    bmod = rows4 % 8
    mask = (rowi == bmod) & (lanei == off)
    rg = jnp.where(mask, rg_s[...], 0.0).sum(axis=3).sum(axis=2) * _SCALE
    an = jnp.where(mask, an_s[...], 0.0).sum(axis=3).sum(axis=2)

    a_lo = an[0:3, :]
    a_hi = an[3:6, :]
    dhw = a_hi - a_lo
    ctr = a_lo + 0.5 * dhw
    pdhw = jnp.exp(rg[3:6, :]) * dhw
    pctr = rg[0:3, :] * dhw + ctr
    mins = jnp.maximum(pctr - 0.5 * pdhw, 0.0)
    row3 = lax.broadcasted_iota(jnp.int32, (3, B), 0)
    lim = jnp.where(row3 == 0, _LIM[0], _LIM[1])
    maxs = jnp.minimum(pctr + 0.5 * pdhw, lim)
    box_out[0:3, :] = mins
    box_out[3:6, :] = maxs


@jax.jit
def _run(cls_flat, reg_t, anc_t):
    mesh = plsc.VectorSubcoreMesh(
        core_axis_name="c", subcore_axis_name="s", num_cores=2, num_subcores=16
    )
    cp = pltpu.CompilerParams()
    if "needs_layout_passes" in pltpu.CompilerParams.__dataclass_fields__:
        cp = dataclasses.replace(cp, needs_layout_passes=False)
    sc_argmax = functools.partial(
        pl.kernel,
        compiler_params=cp,
        out_type=(
            jax.ShapeDtypeStruct((B * L,), jnp.float32),
            jax.ShapeDtypeStruct((B * L,), jnp.int32),
        ),
        mesh=mesh,
        scratch_types=[
            pltpu.VMEM((N,), jnp.float32),
            pltpu.VMEM((L,), jnp.float32),
            pltpu.VMEM((L,), jnp.int32),
        ],
    )(_sc_body)
    scores_full, idx_full = sc_argmax(cls_flat)

    scores, boxes_t = pl.pallas_call(
        _tc_body,
        out_shape=(
            jax.ShapeDtypeStruct((B,), jnp.float32),
            jax.ShapeDtypeStruct((6, B), jnp.float32),
        ),
        in_specs=[
            pl.BlockSpec(memory_space=pltpu.SMEM),
            pl.BlockSpec(memory_space=pltpu.SMEM),
            pl.BlockSpec(memory_space=pl.ANY),
            pl.BlockSpec(memory_space=pl.ANY),
        ],
        out_specs=(
            pl.BlockSpec(memory_space=pltpu.VMEM),
            pl.BlockSpec(memory_space=pltpu.VMEM),
        ),
        scratch_shapes=[
            pltpu.VMEM((6, B, 8, WIN), jnp.float32),
            pltpu.VMEM((6, B, 8, WIN), jnp.float32),
            pltpu.SemaphoreType.DMA,
            pltpu.SemaphoreType.DMA,
        ],
    )(idx_full, scores_full, reg_t, anc_t)

    return scores, jnp.transpose(boxes_t)


def kernel(cls_heads, reg_heads, batch_anchors):
    reg_t = jnp.transpose(reg_heads, (2, 0, 1))
    anc_t = jnp.transpose(batch_anchors, (2, 0, 1))
    return _run(cls_heads.reshape(B * N), reg_t, anc_t)


# speedup vs baseline: 13.3212x; 1.0160x over previous
import dataclasses
import functools

import jax
import jax.numpy as jnp
from jax import lax
from jax.experimental import pallas as pl
from jax.experimental.pallas import tpu as pltpu
from jax.experimental.pallas import tpu_sc as plsc

B = 16
N = 20000
H = N // 2
L = 16
WIN = 128
CH = 5

_LIM = (63.0, 255.0, 255.0)
_SCALE = 0.1


def _scan_phase(c, s, cls_hbm, span_v, stage_v, stage_i, sh_v, sh_i):
    r = 2 * (s % 8) + c
    h = s // 8
    pltpu.sync_copy(cls_hbm.at[pl.ds(r * N + h * H, H)], span_v)

    lane = lax.iota(jnp.int32, L)
    nb = h * H

    def body(i, carry):
        out = []
        for k in range(CH):
            bv, bi = carry[2 * k], carry[2 * k + 1]
            v = span_v[pl.ds((i * CH + k) * L, L)]
            m = v > bv
            out.append(jnp.where(m, v, bv))
            out.append(jnp.where(m, lane + (nb + (i * CH + k) * L), bi))
        return tuple(out)

    init = []
    for k in range(CH):
        init.append(span_v[pl.ds(k * L, L)])
        init.append(lane + (nb + k * L))
    acc = lax.fori_loop(1, (H // L) // CH, body, tuple(init), unroll=1)

    bv, bi = acc[0], acc[1]
    for k in range(1, CH):
        bvk, bik = acc[2 * k], acc[2 * k + 1]
        m = (bvk > bv) | ((bvk == bv) & (bik < bi))
        bv = jnp.where(m, bvk, bv)
        bi = jnp.where(m, bik, bi)

    stage_v[...] = bv
    stage_i[...] = bi
    pltpu.sync_copy(stage_v, sh_v.at[pl.ds(s * L, L)])
    pltpu.sync_copy(stage_i, sh_i.at[pl.ds(s * L, L)])


def _merge_phase(c, s, scores_hbm, idx_hbm, sh_v, sh_i, loc_v, loc_i,
                 sco_v, idx_v):
    pltpu.sync_copy(sh_v, loc_v)
    pltpu.sync_copy(sh_i, loc_i)
    r = 2 * s + c
    bv = loc_v[pl.ds(s * L, L)]
    bi = loc_i[pl.ds(s * L, L)]
    pv = loc_v[pl.ds((s + 8) * L, L)]
    pi = loc_i[pl.ds((s + 8) * L, L)]
    m = (pv > bv) | ((pv == bv) & (pi < bi))
    bv = jnp.where(m, pv, bv)
    bi = jnp.where(m, pi, bi)

    smax = jnp.max(bv)
    cand = jnp.where(bv == smax, bi, jnp.int32(2**31 - 1))
    idx = jnp.min(cand)

    sco_v[...] = jnp.zeros((L,), jnp.float32) + smax
    idx_v[...] = jnp.zeros((L,), jnp.int32) + idx
    pltpu.sync_copy(sco_v, scores_hbm.at[pl.ds(r * L, L)])
    pltpu.sync_copy(idx_v, idx_hbm.at[pl.ds(r * L, L)])


def _sc_body(cls_hbm, scores_hbm, idx_hbm,
             span_v, stage_v, stage_i, sh_v, sh_i, loc_v, loc_i,
             sco_v, idx_v):
    c = lax.axis_index("c")
    s = lax.axis_index("s")
    _scan_phase(c, s, cls_hbm, span_v, stage_v, stage_i, sh_v, sh_i)
    plsc.subcore_barrier()

    @pl.when(s < 8)
    def _():
        _merge_phase(c, s, scores_hbm, idx_hbm, sh_v, sh_i, loc_v, loc_i,
                     sco_v, idx_v)


def _tc_body(idx_smem, sco_smem, reg_any, anc_any, sco_out, box_out,
             rg_s, an_s, sem_r, sem_a):
    cps = []
    for b in range(B):
        ib = idx_smem[b * L]
        w = (ib // WIN) * WIN
        cps.append(pltpu.make_async_copy(
            reg_any.at[:, pl.ds((b // 8) * 8, 8), pl.ds(w, WIN)],
            rg_s.at[:, b], sem_r))
        cps.append(pltpu.make_async_copy(
            anc_any.at[:, pl.ds((b // 8) * 8, 8), pl.ds(w, WIN)],
            an_s.at[:, b], sem_a))
    for cp in cps:
        cp.start()

    rows4 = lax.broadcasted_iota(jnp.int32, (1, B, 1, 1), 1)
    off = jnp.zeros((1, B, 1, 1), jnp.int32)
    lane1 = lax.broadcasted_iota(jnp.int32, (B,), 0)
    sco = jnp.zeros((B,), jnp.float32)
    for b in range(B):
        off = jnp.where(rows4 == b, idx_smem[b * L] % WIN, off)
        sco = jnp.where(lane1 == b, sco_smem[b * L], sco)
    sco_out[...] = sco

    for cp in cps:
        cp.wait()

    rowi = lax.broadcasted_iota(jnp.int32, (1, B, 8, WIN), 2)
    lanei = lax.broadcasted_iota(jnp.int32, (1, B, 8, WIN), 3)
    bmod = rows4 % 8
    mask = (rowi == bmod) & (lanei == off)
    rg = jnp.where(mask, rg_s[...], 0.0).sum(axis=3).sum(axis=2) * _SCALE
    an = jnp.where(mask, an_s[...], 0.0).sum(axis=3).sum(axis=2)

    a_lo = an[0:3, :]
    a_hi = an[3:6, :]
    dhw = a_hi - a_lo
    ctr = a_lo + 0.5 * dhw
    pdhw = jnp.exp(rg[3:6, :]) * dhw
    pctr = rg[0:3, :] * dhw + ctr
    mins = jnp.maximum(pctr - 0.5 * pdhw, 0.0)
    row3 = lax.broadcasted_iota(jnp.int32, (3, B), 0)
    lim = jnp.where(row3 == 0, _LIM[0], _LIM[1])
    maxs = jnp.minimum(pctr + 0.5 * pdhw, lim)
    box_out[0:3, :] = mins
    box_out[3:6, :] = maxs


@jax.jit
def _run(cls_flat, reg_t, anc_t):
    mesh = plsc.VectorSubcoreMesh(
        core_axis_name="c", subcore_axis_name="s", num_cores=2, num_subcores=16
    )
    cp = pltpu.CompilerParams()
    if "needs_layout_passes" in pltpu.CompilerParams.__dataclass_fields__:
        cp = dataclasses.replace(cp, needs_layout_passes=False)
    sc_argmax = functools.partial(
        pl.kernel,
        compiler_params=cp,
        out_type=(
            jax.ShapeDtypeStruct((B * L,), jnp.float32),
            jax.ShapeDtypeStruct((B * L,), jnp.int32),
        ),
        mesh=mesh,
        scratch_types=[
            pltpu.VMEM((H,), jnp.float32),
            pltpu.VMEM((L,), jnp.float32),
            pltpu.VMEM((L,), jnp.int32),
            pltpu.VMEM_SHARED((16 * L,), jnp.float32),
            pltpu.VMEM_SHARED((16 * L,), jnp.int32),
            pltpu.VMEM((16 * L,), jnp.float32),
            pltpu.VMEM((16 * L,), jnp.int32),
            pltpu.VMEM((L,), jnp.float32),
            pltpu.VMEM((L,), jnp.int32),
        ],
    )(_sc_body)
    scores_full, idx_full = sc_argmax(cls_flat)

    scores, boxes_t = pl.pallas_call(
        _tc_body,
        out_shape=(
            jax.ShapeDtypeStruct((B,), jnp.float32),
            jax.ShapeDtypeStruct((6, B), jnp.float32),
        ),
        in_specs=[
            pl.BlockSpec(memory_space=pltpu.SMEM),
            pl.BlockSpec(memory_space=pltpu.SMEM),
            pl.BlockSpec(memory_space=pl.ANY),
            pl.BlockSpec(memory_space=pl.ANY),
        ],
        out_specs=(
            pl.BlockSpec(memory_space=pltpu.VMEM),
            pl.BlockSpec(memory_space=pltpu.VMEM),
        ),
        scratch_shapes=[
            pltpu.VMEM((6, B, 8, WIN), jnp.float32),
            pltpu.VMEM((6, B, 8, WIN), jnp.float32),
            pltpu.SemaphoreType.DMA,
            pltpu.SemaphoreType.DMA,
        ],
    )(idx_full, scores_full, reg_t, anc_t)

    return scores, jnp.transpose(boxes_t)


def kernel(cls_heads, reg_heads, batch_anchors):
    reg_t = jnp.transpose(reg_heads, (2, 0, 1))
    anc_t = jnp.transpose(batch_anchors, (2, 0, 1))
    return _run(cls_heads.reshape(B * N), reg_t, anc_t)
